# baseline (speedup 1.0000x reference)
import jax
import jax.numpy as jnp
from jax import lax
from jax.experimental import pallas as pl
from jax.experimental.pallas import tpu as pltpu
from jax.experimental.pallas import tpu_sc as plsc

ROWS = 1_000_000
COLS = 32
NC = 2
NS = 16
NW = NC * NS
CHUNK = -(-(ROWS // NW) // 8) * 8


def _copy_body(x_hbm, out_hbm):
    wid = lax.axis_index("s") * NC + lax.axis_index("c")
    base = jnp.minimum(wid * CHUNK, ROWS - CHUNK)
    base = pl.multiple_of(base, 8)
    pltpu.sync_copy(x_hbm.at[pl.ds(base, CHUNK)], out_hbm.at[pl.ds(base, CHUNK)])


@jax.jit
def kernel(x):
    return pl.kernel(
        _copy_body,
        out_type=jax.ShapeDtypeStruct((ROWS, COLS), jnp.float32),
        mesh=plsc.VectorSubcoreMesh(core_axis_name="c", subcore_axis_name="s"),
    )(x)


# speedup vs baseline: 14.6151x; 14.6151x over previous
import jax
import jax.numpy as jnp
from jax import lax
from jax.experimental import pallas as pl
from jax.experimental.pallas import tpu as pltpu
from jax.experimental.pallas import tpu_sc as plsc

ROWS = 1_000_000
COLS = 32
WORDS = ROWS * COLS
NC = 2
NS = 16
NW = NC * NS
CW = 32768
T = -(-WORDS // CW)
G = -(-T // NW)
NBUF = 3


def _copy_body(x_hbm, out_hbm, b0, b1, b2, si0, si1, si2, so0, so1, so2):
    bufs = (b0, b1, b2)
    sin = (si0, si1, si2)
    sout = (so0, so1, so2)
    wid = lax.axis_index("s") * NC + lax.axis_index("c")

    def base(i):
        t = jnp.minimum(wid * G + i, T - 1)
        return pl.multiple_of(jnp.minimum(t * CW, WORDS - CW), 8)

    def start_in(i, b):
        pltpu.make_async_copy(x_hbm.at[pl.ds(base(i), CW)], bufs[b], sin[b]).start()

    for g in range(NBUF):
        start_in(g, g)
    for g in range(G):
        b = g % NBUF
        pltpu.make_async_copy(x_hbm.at[pl.ds(base(g), CW)], bufs[b], sin[b]).wait()
        out_cp = pltpu.make_async_copy(bufs[b], out_hbm.at[pl.ds(base(g), CW)], sout[b])
        out_cp.start()
        out_cp.wait()
        if g + NBUF < G:
            start_in(g + NBUF, b)


@jax.jit
def kernel(x):
    flat = pl.kernel(
        _copy_body,
        out_type=jax.ShapeDtypeStruct((WORDS,), jnp.float32),
        mesh=plsc.VectorSubcoreMesh(core_axis_name="c", subcore_axis_name="s"),
        scratch_types=(
            [pltpu.VMEM((CW,), jnp.float32) for _ in range(NBUF)]
            + [pltpu.SemaphoreType.DMA for _ in range(2 * NBUF)]
        ),
    )(x.reshape(WORDS))
    return flat.reshape(ROWS, COLS)
